# baseline (speedup 1.0000x reference)
import functools

import jax
import jax.numpy as jnp
from jax import lax
from jax.experimental import pallas as pl
from jax.experimental.pallas import tpu as pltpu
from jax.experimental.pallas import tpu_sc as plsc

N = 100000
H = 4
K = 16
B = 4096
C = 1.0

NC = 2
NS = 16
NW = NC * NS
LANES = 16

SUP = 16
NSUP = N * H // SUP
HSUP = N // SUP
WPH = NW // H
CH = 784
LASTCH = HSUP - (WPH - 1) * CH
BW = B // NW
NBINS = H * K


def _sc_body(ysup_hbm, lamsup_hbm, idx_hbm,
             hist_out, yb_out, lamb_out,
             ybuf, hist, hist1, hist2, hist3, idxbuf, supidx, ygat, lamgat,
             youtb, loutb, sem_y, sem_l):
    cid = lax.axis_index("c")
    sid = lax.axis_index("s")
    wid = sid * NC + cid

    lane = lax.iota(jnp.int32, LANES)

    gbase = wid * BW
    pltpu.sync_copy(idx_hbm.at[pl.ds(gbase, BW)], idxbuf)
    for i in range(BW // LANES):
        sup = idxbuf[pl.ds(i * LANES, LANES)] // SUP
        for h in range(H):
            supidx[h, pl.ds(i * LANES, LANES)] = sup + h * HSUP
    cps = []
    for h in range(H):
        cps.append(pltpu.async_copy(ysup_hbm.at[supidx.at[h]],
                                    ygat.at[h], sem_y))
        cps.append(pltpu.async_copy(lamsup_hbm.at[supidx.at[h]],
                                    lamgat.at[h], sem_l))

    hmine = wid // WPH
    sub = wid % WPH
    rstart = hmine * HSUP + sub * CH

    @pl.when(sub < WPH - 1)
    def _():
        pltpu.sync_copy(ysup_hbm.at[pl.ds(rstart, CH)], ybuf.at[pl.ds(0, CH)])

    @pl.when(sub == WPH - 1)
    def _():
        pltpu.sync_copy(ysup_hbm.at[pl.ds(rstart, LASTCH)],
                        ybuf.at[pl.ds(0, LASTCH)])

    zeros16 = jnp.zeros((LANES,), jnp.int32)
    for hb in (hist, hist1, hist2, hist3):
        for r in range(NBINS):
            hb[r] = zeros16

    hbase = zeros16 + hmine * K
    ones = jnp.ones((LANES,), jnp.int32)
    nvec = jnp.where(sub == WPH - 1, LASTCH, CH)

    UNROLL = 8
    bufs = (hist, hist1, hist2, hist3)

    def body8(i, carry):
        for j in range(UNROLL):
            v = ybuf[i * UNROLL + j]
            plsc.addupdate_scatter(bufs[j % 4], [hbase + v, lane], ones)
        return carry

    def body1(i, carry):
        v = ybuf[i]
        plsc.addupdate_scatter(hist, [hbase + v, lane], ones)
        return carry

    n8 = nvec // UNROLL
    lax.fori_loop(0, n8, body8, 0)
    lax.fori_loop(n8 * UNROLL, nvec, body1, 0)

    for r in range(NBINS):
        hist[r] = hist[r] + hist1[r] + hist2[r] + hist3[r]

    pltpu.sync_copy(hist, hist_out.at[wid])

    for cp in cps:
        cp.wait()
    for i in range(BW // LANES):
        jvec = lane + i * LANES
        v = idxbuf[pl.ds(i * LANES, LANES)]
        col = v % SUP
        for h in range(H):
            hvec = jnp.full((LANES,), h, jnp.int32)
            yv = plsc.load_gather(ygat, [hvec, jvec, col])
            plsc.store_scatter(youtb, [jvec, hvec], yv)
            lv = plsc.load_gather(lamgat, [hvec, jvec, col])
            plsc.store_scatter(loutb, [jvec, hvec], lv)
    pltpu.sync_copy(youtb, yb_out.at[pl.ds(gbase, BW)])
    pltpu.sync_copy(loutb, lamb_out.at[pl.ds(gbase, BW)])


@functools.cache
def _make_sc_call():
    return pl.kernel(
        _sc_body,
        out_type=[
            jax.ShapeDtypeStruct((NW, NBINS, LANES), jnp.int32),
            jax.ShapeDtypeStruct((B, H), jnp.int32),
            jax.ShapeDtypeStruct((B, H), jnp.float32),
        ],
        mesh=plsc.VectorSubcoreMesh(core_axis_name="c", subcore_axis_name="s"),
        compiler_params=pltpu.CompilerParams(
            needs_layout_passes=False, use_tc_tiling_on_sc=False),
        scratch_types=[
            pltpu.VMEM((CH, SUP), jnp.int32),
            pltpu.VMEM((NBINS, LANES), jnp.int32),
            pltpu.VMEM((NBINS, LANES), jnp.int32),
            pltpu.VMEM((NBINS, LANES), jnp.int32),
            pltpu.VMEM((NBINS, LANES), jnp.int32),
            pltpu.VMEM((BW,), jnp.int32),
            pltpu.VMEM((H, BW), jnp.int32),
            pltpu.VMEM((H, BW, SUP), jnp.int32),
            pltpu.VMEM((H, BW, SUP), jnp.float32),
            pltpu.VMEM((BW, H), jnp.int32),
            pltpu.VMEM((BW, H), jnp.float32),
            pltpu.SemaphoreType.DMA,
            pltpu.SemaphoreType.DMA,
        ],
    )


def _tc_body(x_ref, hist_ref, yb_ref, lamb_ref, out_ref):
    x = x_ref[...]
    counts = jnp.sum(hist_ref[...].astype(jnp.float32), axis=(0, 2))
    base = (C * lax.rsqrt(jnp.sqrt(counts))).reshape(1, NBINS)

    eh = lax.broadcasted_iota(jnp.int32, (H, NBINS), 0)
    ej = lax.broadcasted_iota(jnp.int32, (H, NBINS), 1) // K
    exp_mat = (eh == ej).astype(jnp.float32)

    m = jnp.max(x, axis=1, keepdims=True)
    e = jnp.exp(x - m)
    ssum4 = lax.dot_general(e, exp_mat, (((1,), (1,)), ((), ())),
                            preferred_element_type=jnp.float32)
    lse = lax.dot(jnp.log(ssum4), exp_mat,
                  preferred_element_type=jnp.float32) + m

    ybx = lax.dot(yb_ref[...].astype(jnp.float32), exp_mat,
                  preferred_element_type=jnp.float32)
    lambx = lax.dot(lamb_ref[...], exp_mat, preferred_element_type=jnp.float32)

    kmod = (lax.broadcasted_iota(jnp.int32, (B, NBINS), 1) % K).astype(
        jnp.float32)
    onehot = (kmod == ybx).astype(jnp.float32)
    gated = onehot * jnp.maximum(lse - x - base, 0.0)
    out_ref[...] = (jnp.sum(gated * lambx) * (1.0 / (B * H))).reshape(1, 1)


_tc_call = pl.pallas_call(
    _tc_body,
    out_shape=jax.ShapeDtypeStruct((1, 1), jnp.float32),
)


def kernel(logits, lam, Y, idx):
    Y = Y.astype(jnp.int32)
    idx = idx.astype(jnp.int32)
    ysup = jnp.transpose(Y).reshape(NSUP, SUP)
    lamsup = jnp.transpose(lam).reshape(NSUP, SUP)
    hist, yb, lamb = _make_sc_call()(ysup, lamsup, idx)
    out = _tc_call(logits.reshape(B, H * K), hist, yb, lamb)
    return out[0, 0]


# speedup vs baseline: 1.1179x; 1.1179x over previous
import functools

import jax
import jax.numpy as jnp
from jax import lax
from jax.experimental import pallas as pl
from jax.experimental.pallas import tpu as pltpu
from jax.experimental.pallas import tpu_sc as plsc

N = 100000
H = 4
K = 16
B = 4096
C = 1.0

NC = 2
NS = 16
NW = NC * NS
LANES = 16

SUP = 16
NSUP = N * H // SUP
HSUP = N // SUP
WPH = NW // H
CH = 784
LASTCH = HSUP - (WPH - 1) * CH
BW = B // NW
NBINS = H * K


def _sc_body(ysup_hbm, lamsup_hbm, idx_hbm,
             hist_out, yb_out, lamb_out,
             ybuf, hist, idxbuf, supidx, ygat, lamgat,
             youtb, loutb, sem_y, sem_l):
    cid = lax.axis_index("c")
    sid = lax.axis_index("s")
    wid = sid * NC + cid

    lane = lax.iota(jnp.int32, LANES)

    gbase = wid * BW
    pltpu.sync_copy(idx_hbm.at[pl.ds(gbase, BW)], idxbuf)
    for i in range(BW // LANES):
        sup = idxbuf[pl.ds(i * LANES, LANES)] // SUP
        for h in range(H):
            supidx[h, pl.ds(i * LANES, LANES)] = sup + h * HSUP
    cps = []
    for h in range(H):
        cps.append(pltpu.async_copy(ysup_hbm.at[supidx.at[h]],
                                    ygat.at[h], sem_y))
        cps.append(pltpu.async_copy(lamsup_hbm.at[supidx.at[h]],
                                    lamgat.at[h], sem_l))

    hmine = wid // WPH
    sub = wid % WPH
    rstart = hmine * HSUP + sub * CH

    @pl.when(sub < WPH - 1)
    def _():
        pltpu.sync_copy(ysup_hbm.at[pl.ds(rstart, CH)], ybuf.at[pl.ds(0, CH)])

    @pl.when(sub == WPH - 1)
    def _():
        pltpu.sync_copy(ysup_hbm.at[pl.ds(rstart, LASTCH)],
                        ybuf.at[pl.ds(0, LASTCH)])

    zeros16 = jnp.zeros((LANES,), jnp.int32)
    for r in range(NBINS):
        hist[r] = zeros16

    hbase = zeros16 + hmine * K
    ones = jnp.ones((LANES,), jnp.int32)
    nvec = jnp.where(sub == WPH - 1, LASTCH, CH)

    UNROLL = 8

    def body8(i, carry):
        for j in range(UNROLL):
            v = ybuf[i * UNROLL + j]
            plsc.addupdate_scatter(hist, [hbase + v, lane], ones)
        return carry

    def body1(i, carry):
        v = ybuf[i]
        plsc.addupdate_scatter(hist, [hbase + v, lane], ones)
        return carry

    n8 = nvec // UNROLL
    lax.fori_loop(0, n8, body8, 0)
    lax.fori_loop(n8 * UNROLL, nvec, body1, 0)

    pltpu.sync_copy(hist, hist_out.at[wid])

    for cp in cps:
        cp.wait()
    for i in range(BW // LANES):
        jvec = lane + i * LANES
        v = idxbuf[pl.ds(i * LANES, LANES)]
        col = v % SUP
        for h in range(H):
            yv = plsc.load_gather(ygat, [jnp.full((LANES,), h, jnp.int32),
                                         jvec, col])
            youtb[h, pl.ds(i * LANES, LANES)] = yv
            lv = plsc.load_gather(lamgat, [jnp.full((LANES,), h, jnp.int32),
                                           jvec, col])
            loutb[h, pl.ds(i * LANES, LANES)] = lv
    pltpu.sync_copy(youtb, yb_out.at[:, pl.ds(gbase, BW)])
    pltpu.sync_copy(loutb, lamb_out.at[:, pl.ds(gbase, BW)])


@functools.cache
def _make_sc_call():
    return pl.kernel(
        _sc_body,
        out_type=[
            jax.ShapeDtypeStruct((NW, NBINS, LANES), jnp.int32),
            jax.ShapeDtypeStruct((H, B), jnp.int32),
            jax.ShapeDtypeStruct((H, B), jnp.float32),
        ],
        mesh=plsc.VectorSubcoreMesh(core_axis_name="c", subcore_axis_name="s"),
        compiler_params=pltpu.CompilerParams(
            needs_layout_passes=False, use_tc_tiling_on_sc=False),
        scratch_types=[
            pltpu.VMEM((CH, SUP), jnp.int32),
            pltpu.VMEM((NBINS, LANES), jnp.int32),
            pltpu.VMEM((BW,), jnp.int32),
            pltpu.VMEM((H, BW), jnp.int32),
            pltpu.VMEM((H, BW, SUP), jnp.int32),
            pltpu.VMEM((H, BW, SUP), jnp.float32),
            pltpu.VMEM((H, BW), jnp.int32),
            pltpu.VMEM((H, BW), jnp.float32),
            pltpu.SemaphoreType.DMA,
            pltpu.SemaphoreType.DMA,
        ],
    )


def _tc_body(x_ref, hist_ref, yb_ref, lamb_ref, out_ref):
    x = x_ref[...]
    counts = jnp.sum(hist_ref[...].astype(jnp.float32), axis=(0, 2))
    base = (C * lax.rsqrt(jnp.sqrt(counts))).reshape(1, NBINS)

    eh = lax.broadcasted_iota(jnp.int32, (H, NBINS), 0)
    ej = lax.broadcasted_iota(jnp.int32, (H, NBINS), 1) // K
    exp_mat = (eh == ej).astype(jnp.float32)

    m = jnp.max(x, axis=1, keepdims=True)
    e = jnp.exp(x - m)
    ssum4 = lax.dot_general(e, exp_mat, (((1,), (1,)), ((), ())),
                            preferred_element_type=jnp.float32)
    lse = lax.dot(jnp.log(ssum4), exp_mat,
                  preferred_element_type=jnp.float32) + m

    ybx = lax.dot_general(yb_ref[...].astype(jnp.float32), exp_mat,
                          (((0,), (0,)), ((), ())),
                          preferred_element_type=jnp.float32)
    lambx = lax.dot_general(lamb_ref[...], exp_mat, (((0,), (0,)), ((), ())),
                            preferred_element_type=jnp.float32)

    kmod = (lax.broadcasted_iota(jnp.int32, (B, NBINS), 1) % K).astype(
        jnp.float32)
    onehot = (kmod == ybx).astype(jnp.float32)
    gated = onehot * jnp.maximum(lse - x - base, 0.0)
    out_ref[...] = (jnp.sum(gated * lambx) * (1.0 / (B * H))).reshape(1, 1)


_tc_call = pl.pallas_call(
    _tc_body,
    out_shape=jax.ShapeDtypeStruct((1, 1), jnp.float32),
)


def kernel(logits, lam, Y, idx):
    Y = Y.astype(jnp.int32)
    idx = idx.astype(jnp.int32)
    ysup = jnp.transpose(Y).reshape(NSUP, SUP)
    lamsup = jnp.transpose(lam).reshape(NSUP, SUP)
    hist, yb, lamb = _make_sc_call()(ysup, lamsup, idx)
    out = _tc_call(logits.reshape(B, H * K), hist, yb, lamb)
    return out[0, 0]
